# baseline (speedup 1.0000x reference)
import functools
import math

import jax
import jax.numpy as jnp
from jax import lax
from jax.experimental import pallas as pl
from jax.experimental.pallas import tpu as pltpu
from jax.experimental.pallas import tpu_sc as plsc

D_MODEL = 768
SCALE = math.sqrt(D_MODEL)
NC, NS, LANES = 2, 16, 16
NW = NC * NS
PIECE = 32
NSLOT = 5


def _scale_piece(buf):
    def row_body(r, carry):
        for c in range(D_MODEL // LANES):
            sl = pl.ds(c * LANES, LANES)
            buf[r, sl] = buf[r, sl] * SCALE
        return carry

    lax.fori_loop(0, PIECE, row_body, 0)


def _emb_body(npieces, b_per_w, x_hbm, tab_hbm, out_hbm, idx_v, rows_v, *sems):
    gs, ss = sems[:NSLOT], sems[NSLOT:]
    wid = lax.axis_index("s") * NC + lax.axis_index("c")
    base = wid * b_per_w
    pltpu.sync_copy(x_hbm.at[wid], idx_v)

    def start_gather(k, s):
        pltpu.async_copy(tab_hbm.at[idx_v.at[k]], rows_v.at[s], gs[s])

    def wait_gather(s):
        pltpu.make_async_copy(tab_hbm.at[idx_v.at[0]], rows_v.at[s], gs[s]).wait()

    def start_store(k, s):
        dst = out_hbm.at[pl.ds(base + k * PIECE, PIECE)]
        pltpu.async_copy(rows_v.at[s], dst, ss[s])

    def wait_store(s):
        dst = out_hbm.at[pl.ds(base, PIECE)]
        pltpu.make_async_copy(rows_v.at[s], dst, ss[s]).wait()

    for s in range(NSLOT):
        start_gather(s, s)

    for k in range(npieces):
        s = k % NSLOT
        wait_gather(s)
        _scale_piece(rows_v.at[s])
        start_store(k, s)
        kd = k - 1
        if kd >= 0 and kd + NSLOT < npieces:
            wait_store(kd % NSLOT)
            start_gather(kd + NSLOT, kd % NSLOT)

    for k in range(npieces - NSLOT, npieces):
        wait_store(k % NSLOT)


def kernel(x, embedding_weight):
    orig_shape = x.shape
    b_total = x.size
    b_per_w = b_total // NW
    npieces = b_per_w // PIECE
    x_resh = x.reshape(NW, npieces, PIECE).astype(jnp.int32)

    mesh = plsc.VectorSubcoreMesh(core_axis_name="c", subcore_axis_name="s")
    emb = pl.kernel(
        functools.partial(_emb_body, npieces, b_per_w),
        out_type=jax.ShapeDtypeStruct((b_total, D_MODEL), jnp.float32),
        mesh=mesh,
        scratch_types=[
            pltpu.VMEM((npieces, PIECE), jnp.int32),
            pltpu.VMEM((NSLOT, PIECE, D_MODEL), jnp.float32),
        ] + [pltpu.SemaphoreType.DMA] * (2 * NSLOT),
    )
    out = emb(x_resh, embedding_weight)
    return out.reshape(orig_shape + (D_MODEL,))


# speedup vs baseline: 1.0755x; 1.0755x over previous
import functools
import math

import jax
import jax.numpy as jnp
from jax import lax
from jax.experimental import pallas as pl
from jax.experimental.pallas import tpu as pltpu
from jax.experimental.pallas import tpu_sc as plsc

D_MODEL = 768
SCALE = math.sqrt(D_MODEL)
NC, NS, LANES = 2, 16, 16
NW = NC * NS
CHUNK = 64
NBUF = 2
SPLITS = 2
SUB = CHUNK // SPLITS


def _scale_rows(buf, start, nrows):
    def row_body(r, carry):
        for c in range(D_MODEL // LANES):
            sl = pl.ds(c * LANES, LANES)
            buf[r, sl] = buf[r, sl] * SCALE
        return carry

    lax.fori_loop(start, start + nrows, row_body, 0)


def _emb_body(nchunks, b_per_w, x_hbm, tab_hbm, out_hbm, idx_v, rows_v, *sems):
    gs, ss = sems[:SPLITS * NBUF], sems[SPLITS * NBUF:]
    wid = lax.axis_index("s") * NC + lax.axis_index("c")
    base = wid * b_per_w
    pltpu.sync_copy(x_hbm.at[wid], idx_v)

    def start_gather_half(j, b, h):
        src = tab_hbm.at[idx_v.at[SPLITS * j + h]]
        dst = rows_v.at[b].at[pl.ds(h * SUB, SUB)]
        pltpu.async_copy(src, dst, gs[SPLITS * b + h])

    def wait_gather(b, h):
        dst = rows_v.at[b].at[pl.ds(h * SUB, SUB)]
        pltpu.make_async_copy(tab_hbm.at[idx_v.at[0]], dst, gs[SPLITS * b + h]).wait()

    def start_store_half(j, b, h):
        src = rows_v.at[b].at[pl.ds(h * SUB, SUB)]
        dst = out_hbm.at[pl.ds(base + j * CHUNK + h * SUB, SUB)]
        pltpu.async_copy(src, dst, ss[SPLITS * b + h])

    def wait_store_half(b, h):
        dst = out_hbm.at[pl.ds(base, SUB)]
        src = rows_v.at[b].at[pl.ds(h * SUB, SUB)]
        pltpu.make_async_copy(src, dst, ss[SPLITS * b + h]).wait()

    def piece(j, b, h):
        wait_gather(b, h)
        _scale_rows(rows_v.at[b], h * SUB, SUB)
        start_store_half(j, b, h)

    for b in range(NBUF):
        for h in range(SPLITS):
            start_gather_half(b, b, h)

    ngroups = nchunks // NBUF

    def group_body(g, carry):
        for b in range(NBUF):
            for h in range(SPLITS):
                piece(g * NBUF + b, b, h)
            for h in range(SPLITS):
                wait_store_half(b, h)
                start_gather_half((g + 1) * NBUF + b, b, h)
        return carry

    lax.fori_loop(0, ngroups - 1, group_body, 0)

    g = ngroups - 1
    for b in range(NBUF):
        for h in range(SPLITS):
            piece(g * NBUF + b, b, h)
    for b in range(NBUF):
        for h in range(SPLITS):
            wait_store_half(b, h)


def kernel(x, embedding_weight):
    orig_shape = x.shape
    b_total = x.size
    b_per_w = b_total // NW
    nchunks = b_per_w // CHUNK
    x_resh = x.reshape(NW, SPLITS * nchunks, SUB).astype(jnp.int32)

    mesh = plsc.VectorSubcoreMesh(core_axis_name="c", subcore_axis_name="s")
    emb = pl.kernel(
        functools.partial(_emb_body, nchunks, b_per_w),
        out_type=jax.ShapeDtypeStruct((b_total, D_MODEL), jnp.float32),
        mesh=mesh,
        scratch_types=[
            pltpu.VMEM((SPLITS * nchunks, SUB), jnp.int32),
            pltpu.VMEM((NBUF, CHUNK, D_MODEL), jnp.float32),
        ] + [pltpu.SemaphoreType.DMA] * (2 * SPLITS * NBUF),
    )
    out = emb(x_resh, embedding_weight)
    return out.reshape(orig_shape + (D_MODEL,))
